# baseline (speedup 1.0000x reference)
import functools

import jax
import jax.numpy as jnp
from jax import lax
from jax.experimental import pallas as pl
from jax.experimental.pallas import tpu as pltpu
from jax.experimental.pallas import tpu_sc as plsc

NC = 2
NS = 16
CH = 128
KCH = 20
G = 5


def _lin_in_body(x_ref, w_ref, b_ref, o_ref):
    o_ref[...] = jax.nn.relu(
        jnp.dot(x_ref[...], w_ref[...], preferred_element_type=jnp.float32)
        + b_ref[...])


def _lin_in(x, w, b, rb):
    n, d = x.shape
    hid = w.shape[1]
    return pl.pallas_call(
        _lin_in_body,
        grid=(n // rb,),
        in_specs=[
            pl.BlockSpec((rb, d), lambda i: (i, 0)),
            pl.BlockSpec((d, hid), lambda i: (0, 0)),
            pl.BlockSpec((1, hid), lambda i: (0, 0)),
        ],
        out_specs=pl.BlockSpec((rb, hid), lambda i: (i, 0)),
        out_shape=jax.ShapeDtypeStruct((n, hid), jnp.float32),
    )(x, w, b)


def _msg_body(ea_ref, hs_ref, wk_ref, bk_ref, o_ref):
    ew = jax.nn.relu(
        lax.dot_general(ea_ref[...], wk_ref[...], (((0,), (0,)), ((), ())),
                        preferred_element_type=jnp.float32)
        + bk_ref[...])
    be = ea_ref.shape[1]
    hp = hs_ref[...]
    hs = jnp.concatenate([hp[:, m * 16:(m + 1) * 16] for m in range(8)], axis=0)
    ji = lax.broadcasted_iota(jnp.int32, (16, 256), 1)
    ii = lax.broadcasted_iota(jnp.int32, (16, 256), 0)
    tmat = (ji - (ji // 16) * 16 == ii).astype(jnp.float32)
    h_tile = jnp.dot(hs, tmat, preferred_element_type=jnp.float32)
    prod = ew * h_tile
    jr = lax.broadcasted_iota(jnp.int32, (256, 32), 0) // 16
    oc = lax.broadcasted_iota(jnp.int32, (256, 32), 1)
    sel = (jr == oc).astype(jnp.float32)
    msg = jnp.dot(prod, sel, preferred_element_type=jnp.float32)
    cone = (lax.broadcasted_iota(jnp.int32, (1, 32), 1) == 16).astype(jnp.float32)
    msg = msg + cone
    q = be // 4
    o_ref[...] = jnp.concatenate(
        [msg[m * q:(m + 1) * q, :] for m in range(4)], axis=1)


def _edge_messages(ea_t, h_src_p, wk_perm, bk_perm, be, eh, blk_off):
    return pl.pallas_call(
        _msg_body,
        grid=(eh // be,),
        in_specs=[
            pl.BlockSpec((ea_t.shape[0], be), lambda i: (0, i + blk_off)),
            pl.BlockSpec((be // 8, 128), lambda i: (i, 0)),
            pl.BlockSpec((ea_t.shape[0], 256), lambda i: (0, 0)),
            pl.BlockSpec((1, 256), lambda i: (0, 0)),
        ],
        out_specs=pl.BlockSpec((be // 4, 128), lambda i: (i, 0)),
        out_shape=jax.ShapeDtypeStruct((eh // 4, 128), jnp.float32),
    )(ea_t, h_src_p, wk_perm, bk_perm)


def _sc_gather_body(nch, coff, h_hbm, src2d_hbm, out_hbm, idx_v, rows_v, sem, wsem):
    tid = lax.axis_index("s") * NC + lax.axis_index("c")
    cbase = tid * KCH
    nj = jnp.clip(nch - cbase, 0, KCH)
    ng = nj // G

    def ldfire(gi):
        pltpu.sync_copy(src2d_hbm.at[pl.ds(coff + cbase + gi * G, G)],
                        idx_v.at[pl.ds(gi * G, G)])
        for b in range(G):
            j = gi * G + b
            pltpu.async_copy(
                h_hbm.at[idx_v.at[j]], rows_v.at[pl.ds(j * CH, CH)], sem)

    @pl.when(ng > 0)
    def _():
        ldfire(0)

    def group(gi, _):
        @pl.when(gi + 1 < ng)
        def _():
            ldfire(gi + 1)
        for _b in range(G):
            pltpu.make_async_copy(
                h_hbm.at[idx_v.at[0]], rows_v.at[pl.ds(0, CH)], sem).wait()
        pltpu.async_copy(
            rows_v.at[pl.ds(gi * G * CH, G * CH)],
            out_hbm.at[pl.ds((cbase + gi * G) * CH, G * CH)], wsem)
        return 0
    lax.fori_loop(0, ng, group, 0)

    def drain(gi, _):
        pltpu.make_async_copy(
            rows_v.at[pl.ds(0, G * CH)],
            out_hbm.at[pl.ds(cbase * CH, G * CH)], wsem).wait()
        return 0
    lax.fori_loop(0, ng, drain, 0)


def _sc_gather(h, src2d, nch, coff):
    eh = nch * CH
    mesh = plsc.VectorSubcoreMesh(
        core_axis_name="c", subcore_axis_name="s",
        num_cores=NC, num_subcores=NS)
    body = functools.partial(_sc_gather_body, nch, coff)
    fn = pl.kernel(
        body,
        out_type=jax.ShapeDtypeStruct((eh, 16), jnp.float32),
        mesh=mesh,
        compiler_params=pltpu.CompilerParams(use_tc_tiling_on_sc=False),
        scratch_types=[
            pltpu.VMEM((KCH, CH), jnp.int32),
            pltpu.VMEM((KCH * CH, 16), jnp.float32),
            pltpu.SemaphoreType.DMA,
            pltpu.SemaphoreType.DMA,
        ],
    )
    return fn(h, src2d)


def _sc_scatter_body(nch, coff, npad, msg_hbm, dst2d_hbm, parts_hbm,
                     idx_v, v0, v1, z_v, acc_sh, sem0, sem1, ss0, ss1):
    cid = lax.axis_index("c")
    sid = lax.axis_index("s")
    tid = cid * NS + sid
    cbase = tid * KCH
    nj = jnp.clip(nch - cbase, 0, KCH)
    rps = npad // NS

    def zrow(r, _):
        z_v[r, pl.ds(0, 16)] = jnp.zeros((16,), jnp.float32)
        z_v[r, pl.ds(16, 16)] = jnp.zeros((16,), jnp.float32)
        return 0
    lax.fori_loop(0, CH, zrow, 0)

    def zshared(k2, _):
        pltpu.async_copy(z_v, acc_sh.at[pl.ds(sid * rps + k2 * CH, CH)], ss0)
        return 0
    lax.fori_loop(0, rps // CH, zshared, 0)

    def ldidx(gi, _):
        pltpu.async_copy(dst2d_hbm.at[pl.ds(coff + cbase + gi * G, G)],
                         idx_v.at[pl.ds(gi * G, G)], ss1)
        return 0
    lax.fori_loop(0, nj // G, ldidx, 0)

    def zdrain(k2, _):
        pltpu.make_async_copy(z_v, acc_sh.at[pl.ds(0, CH)], ss0).wait()
        return 0
    lax.fori_loop(0, rps // CH, zdrain, 0)

    def idrain(gi, _):
        pltpu.make_async_copy(dst2d_hbm.at[pl.ds(0, G)],
                              idx_v.at[pl.ds(0, G)], ss1).wait()
        return 0
    lax.fori_loop(0, nj // G, idrain, 0)
    plsc.subcore_barrier()

    @pl.when(nj > 0)
    def _():
        pltpu.async_copy(msg_hbm.at[pl.ds(cbase * CH, CH)], v0, sem0)

    @pl.when(nj > 1)
    def _():
        pltpu.async_copy(msg_hbm.at[pl.ds((cbase + 1) * CH, CH)], v1, sem1)

    def pair(j2, _):
        j = 2 * j2
        pltpu.make_async_copy(msg_hbm.at[pl.ds(cbase * CH, CH)], v0, sem0).wait()
        pltpu.async_copy(v0, acc_sh.at[idx_v.at[j]], ss0, add=True)
        pltpu.make_async_copy(msg_hbm.at[pl.ds(cbase * CH, CH)], v1, sem1).wait()
        pltpu.async_copy(v1, acc_sh.at[idx_v.at[j + 1]], ss1, add=True)
        pltpu.make_async_copy(v0, acc_sh.at[idx_v.at[j]], ss0).wait()

        @pl.when(j + 2 < nj)
        def _():
            pltpu.async_copy(
                msg_hbm.at[pl.ds((cbase + j + 2) * CH, CH)], v0, sem0)
        pltpu.make_async_copy(v1, acc_sh.at[idx_v.at[j + 1]], ss1).wait()

        @pl.when(j + 3 < nj)
        def _():
            pltpu.async_copy(
                msg_hbm.at[pl.ds((cbase + j + 3) * CH, CH)], v1, sem1)
        return 0
    lax.fori_loop(0, nj // 2, pair, 0)

    @pl.when(nj % 2 == 1)
    def _():
        j = nj - 1
        pltpu.make_async_copy(msg_hbm.at[pl.ds(cbase * CH, CH)], v0, sem0).wait()
        pltpu.async_copy(v0, acc_sh.at[idx_v.at[j]], ss0, add=True)
        pltpu.make_async_copy(v0, acc_sh.at[idx_v.at[j]], ss0).wait()
    plsc.subcore_barrier()

    pltpu.sync_copy(acc_sh.at[pl.ds(sid * rps, rps)],
                    parts_hbm.at[cid, pl.ds(sid * rps, rps)])


def _sc_scatter(msg32, dst2d, nch, coff, npad):
    mesh = plsc.VectorSubcoreMesh(
        core_axis_name="c", subcore_axis_name="s",
        num_cores=NC, num_subcores=NS)
    body = functools.partial(_sc_scatter_body, nch, coff, npad)
    fn = pl.kernel(
        body,
        out_type=jax.ShapeDtypeStruct((NC, npad, 32), jnp.float32),
        mesh=mesh,
        compiler_params=pltpu.CompilerParams(use_tc_tiling_on_sc=False),
        scratch_types=[
            pltpu.VMEM((KCH, CH), jnp.int32),
            pltpu.VMEM((CH, 32), jnp.float32),
            pltpu.VMEM((CH, 32), jnp.float32),
            pltpu.VMEM((CH, 32), jnp.float32),
            pltpu.VMEM_SHARED((npad, 32), jnp.float32),
            pltpu.SemaphoreType.DMA,
            pltpu.SemaphoreType.DMA,
            pltpu.SemaphoreType.DMA,
            pltpu.SemaphoreType.DMA,
        ],
    )
    return fn(msg32, dst2d)


def _final_body(pa_ref, pb_ref, h_ref, wr_ref, bc_ref, wmu_ref, bmu_ref,
                wlv_ref, blv_ref, mu_ref, lv_ref):
    p = pa_ref[0] + pa_ref[1] + pb_ref[0] + pb_ref[1]
    cnt = jnp.maximum(p[:, 16:17], 1.0)
    agg = p[:, :16] / cnt
    h2 = jax.nn.relu(
        agg
        + jnp.dot(h_ref[...], wr_ref[...], preferred_element_type=jnp.float32)
        + bc_ref[...])
    mu_ref[...] = jnp.dot(h2, wmu_ref[...],
                          preferred_element_type=jnp.float32) + bmu_ref[...]
    lv_ref[...] = jnp.dot(h2, wlv_ref[...],
                          preferred_element_type=jnp.float32) + blv_ref[...]


def _final(pa, pb, h, wr, bc, wmu, bmu, wlv, blv, rb):
    n = h.shape[0]
    lat = wmu.shape[1]
    return pl.pallas_call(
        _final_body,
        grid=(n // rb,),
        in_specs=[
            pl.BlockSpec((2, rb, 32), lambda i: (0, i, 0)),
            pl.BlockSpec((2, rb, 32), lambda i: (0, i, 0)),
            pl.BlockSpec((rb, 16), lambda i: (i, 0)),
            pl.BlockSpec((16, 16), lambda i: (0, 0)),
            pl.BlockSpec((1, 16), lambda i: (0, 0)),
            pl.BlockSpec((16, lat), lambda i: (0, 0)),
            pl.BlockSpec((1, lat), lambda i: (0, 0)),
            pl.BlockSpec((16, lat), lambda i: (0, 0)),
            pl.BlockSpec((1, lat), lambda i: (0, 0)),
        ],
        out_specs=[
            pl.BlockSpec((rb, lat), lambda i: (i, 0)),
            pl.BlockSpec((rb, lat), lambda i: (i, 0)),
        ],
        out_shape=[
            jax.ShapeDtypeStruct((n, lat), jnp.float32),
            jax.ShapeDtypeStruct((n, lat), jnp.float32),
        ],
    )(pa, pb, h, wr, bc, wmu, bmu, wlv, blv)


def kernel(x, edge_index, edge_attr, W_in, b_in, Wk, bk, Wr, b_conv,
           Wmu, bmu, Wlv, blv):
    n, in_dim = x.shape
    e = edge_index.shape[1]
    ea_dim = edge_attr.shape[1]
    hid = W_in.shape[1]
    lat = Wmu.shape[1]

    npad = NS * CH * (-(-n // (NS * CH)))
    be = 3200
    nb = e // be
    eh = e // 2
    nch_h = eh // CH

    src = (edge_index[0].reshape(nb, 8, be // 8).transpose(0, 2, 1)
           .reshape(e // CH, CH))
    dst = (edge_index[1].reshape(nb, 4, be // 4).transpose(0, 2, 1)
           .reshape(e // CH, CH))
    wk_perm = Wk.reshape(ea_dim, hid, hid).transpose(0, 2, 1).reshape(ea_dim, hid * hid)
    bk_perm = bk.reshape(hid, hid).T.reshape(1, hid * hid)
    ea_t = edge_attr.T

    h = _lin_in(x, W_in, b_in.reshape(1, hid), rb=2000)

    hs_a = _sc_gather(h, src, nch_h, 0)
    hs_b = _sc_gather(h, src, nch_h, nch_h)
    msg_a = _edge_messages(ea_t, hs_a.reshape(eh * hid // 128, 128),
                           wk_perm, bk_perm, be, eh, 0)
    msg_b = _edge_messages(ea_t, hs_b.reshape(eh * hid // 128, 128),
                           wk_perm, bk_perm, be, eh, eh // be)
    parts_a = _sc_scatter(msg_a.reshape(eh, 32), dst, nch_h, 0, npad)
    parts_b = _sc_scatter(msg_b.reshape(eh, 32), dst, nch_h, nch_h, npad)
    mu, logvar = _final(parts_a, parts_b, h, Wr, b_conv.reshape(1, hid),
                        Wmu, bmu.reshape(1, lat), Wlv, blv.reshape(1, lat),
                        rb=2000)
    return (mu, logvar)
